# Initial kernel scaffold; baseline (speedup 1.0000x reference)
import jax
import jax.numpy as jnp
from jax.experimental import pallas as pl


def kernel(timestamps, knots):
    raise NotImplementedError("write your pallas kernel here")


# speedup vs baseline: 11.2637x; 11.2637x over previous
import functools

import numpy as np
import jax
import jax.numpy as jnp
from jax import lax
from jax.experimental import pallas as pl
from jax.experimental.pallas import tpu as pltpu
from jax.experimental.pallas import tpu_sc as plsc

N_Q = 2097152
N_KNOTS = 100001
_LO = np.float32(0.0 + 1e-4)
_HI = np.float32(1e-05 * (N_KNOTS - 1) - 1e-4)
_IV = np.float32(1e-05)
_PI = np.float32(np.pi)
_PI_2 = np.float32(np.pi / 2)

NC, NS, L = 2, 16, 16
NW = NC * NS
Q_PER_W = N_Q // NW
CHUNK = 2048
N_CHUNKS = Q_PER_W // CHUNK
G = 128
NG = CHUNK // G

_ATAN_C = [0.99999994, -0.33332935, 0.19990744, -0.1419258, 0.10607158,
           -0.07447012, 0.04214998, -0.015803704, 0.0027982248]
_SINC_C = [1.0, -0.16666666, 0.00833332, -0.00019840486, 2.7535289e-06,
           -2.472641e-08, 1.3612299e-10]
_COS_C = [1.0, -0.5, 0.041666664, -0.001388886, 2.4800507e-05,
          -2.753439e-07, 2.0602127e-09, -9.722127e-12]


def _horner(coefs, z):
    acc = jnp.full((L,), np.float32(coefs[-1]), jnp.float32)
    for c in coefs[-2::-1]:
        acc = acc * z + np.float32(c)
    return acc


def _rsqrt(x):
    i = plsc.bitcast(x, jnp.int32)
    y = plsc.bitcast(jnp.int32(0x5F3759DF) - lax.shift_right_logical(i, 1),
                     jnp.float32)
    for _ in range(3):
        y = y * (np.float32(1.5) - np.float32(0.5) * x * y * y)
    return y


def _quat_mul16(ax, ay, az, aw, bx, by, bz, bw):
    x = aw * bx + ax * bw + ay * bz - az * by
    y = aw * by - ax * bz + ay * bw + az * bx
    z = aw * bz + ax * by - ay * bx + az * bw
    w = aw * bw - ax * bx - ay * by - az * bz
    return x, y, z, w


def _spline_body(ts_hbm, pair_hbm, out_hbm, ts_v, idx_v, u_v, rows_v, out_v, gsem):
    wid = lax.axis_index("s") * NC + lax.axis_index("c")
    base_w = wid * Q_PER_W

    def chunk_body(ci, carry):
        base = base_w + ci * CHUNK
        pltpu.sync_copy(ts_hbm.at[pl.ds(base, CHUNK)], ts_v)

        def pass_a(i, c):
            t = ts_v[pl.ds(i * L, L)]
            tc = jnp.minimum(jnp.maximum(t, _LO), _HI)
            norm = tc / _IV
            st = norm.astype(jnp.int32)
            u_v[pl.ds(i * L, L)] = norm - st.astype(jnp.float32)
            idx_v[pl.ds(i * L, L)] = st
            return c

        lax.fori_loop(0, CHUNK // L, pass_a, 0)

        copies = []
        for g in range(NG):
            cp = pltpu.make_async_copy(
                pair_hbm.at[idx_v.at[pl.ds(g * G, G)]],
                rows_v.at[pl.ds(g * G, G)], gsem)
            cp.start()
            copies.append(cp)
        for cp in copies:
            cp.wait()

        def pass_b(i, c):
            r0 = i * L
            rid = lax.broadcasted_iota(jnp.int32, (L,), 0) + r0

            def col(cc):
                idxc = jnp.full((L,), cc, jnp.int32)
                return plsc.load_gather(rows_v, [rid, idxc])

            u = u_v[pl.ds(r0, L)]
            t0x, t0y, t0z = col(0), col(1), col(2)
            q0x, q0y, q0z, q0w = col(3), col(4), col(5), col(6)
            t1x, t1y, t1z = col(8), col(9), col(10)
            q1x, q1y, q1z, q1w = col(11), col(12), col(13), col(14)

            ox = t0x + u * (t1x - t0x)
            oy = t0y + u * (t1y - t0y)
            oz = t0z + u * (t1z - t0z)

            rx, ry, rz, rw = _quat_mul16(-q0x, -q0y, -q0z, q0w,
                                         q1x, q1y, q1z, q1w)
            vv = rx * rx + ry * ry + rz * rz + np.float32(1e-30)
            rn = _rsqrt(vv)
            n = vv * rn
            wa = jnp.abs(rw)
            mn = jnp.minimum(n, wa)
            mx = jnp.maximum(n, wa)
            t = mn / mx
            a = t * _horner(_ATAN_C, t * t)
            a = jnp.where(n > wa, _PI_2 - a, a)
            a = jnp.where(rw < np.float32(0.0), _PI - a, a)
            angle = np.float32(2.0) * a
            scale = angle * rn
            us = u * scale
            px, py, pz = rx * us, ry * us, rz * us
            half = np.float32(0.5) * (u * angle)
            h2 = half * half
            s = np.float32(0.5) * _horner(_SINC_C, h2)
            wq = _horner(_COS_C, h2)
            qx, qy, qz, qw = _quat_mul16(q0x, q0y, q0z, q0w,
                                         px * s, py * s, pz * s, wq)

            for cc, val in ((0, ox), (1, oy), (2, oz),
                            (3, qx), (4, qy), (5, qz), (6, qw)):
                plsc.store_scatter(out_v, [rid, jnp.full((L,), cc, jnp.int32)], val)
            return c

        lax.fori_loop(0, CHUNK // L, pass_b, 0)
        pltpu.sync_copy(out_v, out_hbm.at[pl.ds(base, CHUNK)])
        return carry

    lax.fori_loop(0, N_CHUNKS, chunk_body, 0)


_spline_call = functools.partial(
    pl.kernel,
    mesh=plsc.VectorSubcoreMesh(core_axis_name="c", subcore_axis_name="s"),
    compiler_params=pltpu.CompilerParams(needs_layout_passes=False,
                                         use_tc_tiling_on_sc=False),
    out_type=jax.ShapeDtypeStruct((N_Q, 7), jnp.float32),
    scratch_types=[
        pltpu.VMEM((CHUNK,), jnp.float32),
        pltpu.VMEM((CHUNK,), jnp.int32),
        pltpu.VMEM((CHUNK,), jnp.float32),
        pltpu.VMEM((CHUNK, 16), jnp.float32),
        pltpu.VMEM((CHUNK, 7), jnp.float32),
        pltpu.SemaphoreType.DMA,
    ],
)(_spline_body)


def kernel(timestamps, knots):
    pair = jnp.concatenate([
        jnp.pad(knots[:-1], ((0, 0), (0, 1))),
        jnp.pad(knots[1:], ((0, 0), (0, 1))),
    ], axis=1)
    return _spline_call(timestamps, pair)


# speedup vs baseline: 11.6047x; 1.0303x over previous
import functools

import numpy as np
import jax
import jax.numpy as jnp
from jax import lax
from jax.experimental import pallas as pl
from jax.experimental.pallas import tpu as pltpu
from jax.experimental.pallas import tpu_sc as plsc

N_Q = 2097152
N_KNOTS = 100001
_LO = np.float32(0.0 + 1e-4)
_HI = np.float32(1e-05 * (N_KNOTS - 1) - 1e-4)
_IV = np.float32(1e-05)
_PI = np.float32(np.pi)
_PI_2 = np.float32(np.pi / 2)

NC, NS, L = 2, 16, 16
NW = NC * NS
Q_PER_W = N_Q // NW
CHUNK = 2048
N_CHUNKS = Q_PER_W // CHUNK
G = 128
NG = CHUNK // G

_ATAN_C = [0.9999997615814209, -0.3333137035369873, 0.19963355362415314,
           -0.1399170607328415, 0.098538339138031, -0.05880045145750046,
           0.023868374526500702, -0.004610803909599781]
_SINC_C = [0.9999998807907104, -0.16666607558727264, 0.008332732133567333,
           -0.00019816691929008812, 2.7083260647486895e-06,
           -2.069596938270024e-08]
_COS_C = [1.0, -0.49999985098838806, 0.041666463017463684,
          -0.0013887732056900859, 2.4769053197815083e-05,
          -2.707544979330123e-07, 1.7243751981865785e-09]


def _horner(coefs, z):
    acc = jnp.full((L,), np.float32(coefs[-1]), jnp.float32)
    for c in coefs[-2::-1]:
        acc = acc * z + np.float32(c)
    return acc


def _rsqrt(x):
    i = plsc.bitcast(x, jnp.int32)
    y = plsc.bitcast(jnp.int32(0x5F3759DF) - lax.shift_right_logical(i, 1),
                     jnp.float32)
    for _ in range(2):
        y = y * (np.float32(1.5) - np.float32(0.5) * x * y * y)
    return y


def _quat_mul16(ax, ay, az, aw, bx, by, bz, bw):
    x = aw * bx + ax * bw + ay * bz - az * by
    y = aw * by - ax * bz + ay * bw + az * bx
    z = aw * bz + ax * by - ay * bx + az * bw
    w = aw * bw - ax * bx - ay * by - az * bz
    return x, y, z, w


def _spline_body(ts_hbm, pair_hbm, out_hbm, ts_v, idx_v, u_v, rows_v, out_v, gsem):
    wid = lax.axis_index("s") * NC + lax.axis_index("c")
    base_w = wid * Q_PER_W

    def chunk_body(ci, carry):
        base = base_w + ci * CHUNK
        pltpu.sync_copy(ts_hbm.at[pl.ds(base, CHUNK)], ts_v)

        def pass_a(i, c):
            t = ts_v[pl.ds(i * L, L)]
            tc = jnp.minimum(jnp.maximum(t, _LO), _HI)
            norm = tc / _IV
            st = norm.astype(jnp.int32)
            u_v[pl.ds(i * L, L)] = norm - st.astype(jnp.float32)
            idx_v[pl.ds(i * L, L)] = st
            return c

        lax.fori_loop(0, CHUNK // L, pass_a, 0)

        copies = []
        for g in range(NG):
            cp = pltpu.make_async_copy(
                pair_hbm.at[idx_v.at[pl.ds(g * G, G)]],
                rows_v.at[pl.ds(g * G, G)], gsem)
            cp.start()
            copies.append(cp)
        for cp in copies:
            cp.wait()

        def interp16(r0):
            rid = lax.broadcasted_iota(jnp.int32, (L,), 0) + r0

            def col(cc):
                idxc = jnp.full((L,), cc, jnp.int32)
                return plsc.load_gather(rows_v, [rid, idxc])

            u = u_v[pl.ds(r0, L)]
            t0x, t0y, t0z = col(0), col(1), col(2)
            q0x, q0y, q0z, q0w = col(3), col(4), col(5), col(6)
            t1x, t1y, t1z = col(8), col(9), col(10)
            q1x, q1y, q1z, q1w = col(11), col(12), col(13), col(14)

            ox = t0x + u * (t1x - t0x)
            oy = t0y + u * (t1y - t0y)
            oz = t0z + u * (t1z - t0z)

            rx, ry, rz, rw = _quat_mul16(-q0x, -q0y, -q0z, q0w,
                                         q1x, q1y, q1z, q1w)
            vv = rx * rx + ry * ry + rz * rz + np.float32(1e-30)
            rn = _rsqrt(vv)
            n = vv * rn
            wa = jnp.abs(rw)
            mn = jnp.minimum(n, wa)
            mx = jnp.maximum(n, wa)
            t = mn / mx
            a = t * _horner(_ATAN_C, t * t)
            a = jnp.where(n > wa, _PI_2 - a, a)
            a = jnp.where(rw < np.float32(0.0), _PI - a, a)
            angle = np.float32(2.0) * a
            scale = angle * rn
            us = u * scale
            px, py, pz = rx * us, ry * us, rz * us
            half = np.float32(0.5) * (u * angle)
            h2 = half * half
            s = np.float32(0.5) * _horner(_SINC_C, h2)
            wq = _horner(_COS_C, h2)
            qx, qy, qz, qw = _quat_mul16(q0x, q0y, q0z, q0w,
                                         px * s, py * s, pz * s, wq)

            for cc, val in ((0, ox), (1, oy), (2, oz),
                            (3, qx), (4, qy), (5, qz), (6, qw)):
                plsc.store_scatter(out_v, [rid, jnp.full((L,), cc, jnp.int32)], val)

        def pass_b(i, c):
            interp16(i * (2 * L))
            interp16(i * (2 * L) + L)
            return c

        lax.fori_loop(0, CHUNK // (2 * L), pass_b, 0)
        pltpu.sync_copy(out_v, out_hbm.at[pl.ds(base, CHUNK)])
        return carry

    lax.fori_loop(0, N_CHUNKS, chunk_body, 0)


_spline_call = functools.partial(
    pl.kernel,
    mesh=plsc.VectorSubcoreMesh(core_axis_name="c", subcore_axis_name="s"),
    compiler_params=pltpu.CompilerParams(needs_layout_passes=False,
                                         use_tc_tiling_on_sc=False),
    out_type=jax.ShapeDtypeStruct((N_Q, 7), jnp.float32),
    scratch_types=[
        pltpu.VMEM((CHUNK,), jnp.float32),
        pltpu.VMEM((CHUNK,), jnp.int32),
        pltpu.VMEM((CHUNK,), jnp.float32),
        pltpu.VMEM((CHUNK, 16), jnp.float32),
        pltpu.VMEM((CHUNK, 7), jnp.float32),
        pltpu.SemaphoreType.DMA,
    ],
)(_spline_body)


def kernel(timestamps, knots):
    pair = jnp.concatenate([
        jnp.pad(knots[:-1], ((0, 0), (0, 1))),
        jnp.pad(knots[1:], ((0, 0), (0, 1))),
    ], axis=1)
    return _spline_call(timestamps, pair)
